# baseline (speedup 1.0000x reference)
import functools

import jax
import jax.numpy as jnp
from jax import lax
from jax.experimental import pallas as pl
from jax.experimental.pallas import tpu as pltpu
from jax.experimental.pallas import tpu_sc as plsc

_VOCAB = 1000000
_EMBED = 64
_B = 4096
_L = 50
_WIN = 3
_HID = 1024
_NCLS = 1000

_NIDX = _B * _L
_GW = 128
_BB = 256
_FLATW = (_L + _WIN - 1) * _EMBED
_PADW = (_L + 2 * (_WIN - 1)) * _EMBED


_NW = 32
_PAIRS = _NIDX // 2
_PPW = _PAIRS // _NW
_PSTEP = 128
_NSTEP = _PPW // _PSTEP


def _sc_gather(table, idx):
    mesh = plsc.VectorSubcoreMesh(core_axis_name="c", subcore_axis_name="s")

    @functools.partial(
        pl.kernel,
        out_type=jax.ShapeDtypeStruct((_PAIRS, 2 * _EMBED), jnp.float32),
        mesh=mesh,
        compiler_params=pltpu.CompilerParams(use_tc_tiling_on_sc=False),
        scratch_types=[
            pltpu.VMEM((_PPW,), jnp.int32),
            pltpu.VMEM((_PPW,), jnp.int32),
            pltpu.VMEM((_PSTEP, _EMBED), jnp.float32),
            pltpu.VMEM((_PSTEP, _EMBED), jnp.float32),
            pltpu.SemaphoreType.DMA,
            pltpu.SemaphoreType.DMA,
        ],
    )
    def gather_kernel(
        table_hbm, ie_hbm, io_hbm, out_hbm, ie_v, io_v, se, so, sem0, sem1
    ):
        wid = lax.axis_index("s") * 2 + lax.axis_index("c")
        base = wid * _PPW
        pltpu.sync_copy(ie_hbm.at[pl.ds(base, _PPW)], ie_v)
        pltpu.sync_copy(io_hbm.at[pl.ds(base, _PPW)], io_v)

        @pl.loop(0, _NSTEP)
        def _(s):
            off = s * _PSTEP
            ge = pltpu.async_copy(
                table_hbm.at[ie_v.at[pl.ds(off, _PSTEP)]], se, sem0
            )
            go = pltpu.async_copy(
                table_hbm.at[io_v.at[pl.ds(off, _PSTEP)]], so, sem1
            )
            ge.wait()
            go.wait()
            we = pltpu.async_copy(
                se,
                out_hbm.at[pl.ds(base + off, _PSTEP), pl.ds(0, _EMBED)],
                sem0,
            )
            wo = pltpu.async_copy(
                so,
                out_hbm.at[pl.ds(base + off, _PSTEP), pl.ds(_EMBED, _EMBED)],
                sem1,
            )
            we.wait()
            wo.wait()

    return gather_kernel(table, idx[0], idx[1])


def _mlp_body(emb_ref, r0_ref, w1_ref, b1_ref, w2_ref, b2_ref, out_ref, p_ref):
    r0 = jnp.broadcast_to(r0_ref[...], (_BB, _EMBED))
    p_ref[:, : _EMBED] = r0
    p_ref[:, _EMBED : 2 * _EMBED] = r0
    p_ref[:, 2 * _EMBED : 2 * _EMBED + _L * _EMBED] = emb_ref[...]
    p_ref[:, _PADW - 2 * _EMBED : _PADW - _EMBED] = r0
    p_ref[:, _PADW - _EMBED :] = r0
    p = p_ref[...]
    flat = jnp.maximum(
        jnp.maximum(p[:, :_FLATW], p[:, _EMBED : _EMBED + _FLATW]),
        p[:, 2 * _EMBED : 2 * _EMBED + _FLATW],
    )
    h = jnp.dot(
        flat.astype(jnp.bfloat16), w1_ref[...], preferred_element_type=jnp.float32
    ) + b1_ref[...]
    h = jnp.maximum(h, 0.0).astype(jnp.bfloat16)
    out_ref[...] = jnp.dot(
        h, w2_ref[...], preferred_element_type=jnp.float32
    ) + b2_ref[...]


def _tc_mlp(emb2d, row0, w1, b1, w2, b2):
    grid = (_B // _BB,)
    return pl.pallas_call(
        _mlp_body,
        grid=grid,
        in_specs=[
            pl.BlockSpec((_BB, _L * _EMBED), lambda i: (i, 0)),
            pl.BlockSpec((1, _EMBED), lambda i: (0, 0)),
            pl.BlockSpec((_FLATW, _HID), lambda i: (0, 0)),
            pl.BlockSpec((1, _HID), lambda i: (0, 0)),
            pl.BlockSpec((_HID, _NCLS), lambda i: (0, 0)),
            pl.BlockSpec((1, _NCLS), lambda i: (0, 0)),
        ],
        out_specs=pl.BlockSpec((_BB, _NCLS), lambda i: (i, 0)),
        out_shape=jax.ShapeDtypeStruct((_B, _NCLS), jnp.float32),
        scratch_shapes=[pltpu.VMEM((_BB, _PADW), jnp.float32)],
    )(emb2d, row0, w1, b1, w2, b2)


def kernel(x, table, W1, b1, W2, b2):
    idx = x.astype(jnp.int32).reshape(_NIDX // 2, 2).T
    emb = _sc_gather(table, idx)
    emb2d = emb.reshape(_B, _L * _EMBED)
    row0 = lax.slice(table, (0, 0), (1, _EMBED))
    w1 = W1.astype(jnp.bfloat16)
    w2 = W2.astype(jnp.bfloat16)
    return _tc_mlp(
        emb2d, row0, w1, b1.reshape(1, _HID), w2, b2.reshape(1, _NCLS)
    )
